# baseline (speedup 1.0000x reference)
import jax
import jax.numpy as jnp
from jax.experimental import pallas as pl
from jax.experimental.pallas import tpu as pltpu

_EPS = 1e-5
_NPTS = 20
_C = 9
_H = 64
_OUT = 128
_RB = 2048
_NBLK = 16384 // _RB


def _mega_kernel(poly_ref, mask_ref, w1_ref, w2a_ref, w2b_ref,
                 w3_ref, w4_ref, b4_ref, w5_ref, b5_ref, g1_ref, be1_ref,
                 g2_ref, be2_ref, g3_ref, be3_ref, out_ref,
                 x2_scr, pb_scr, w1s_scr, w3s_scr, w4s_scr, gram1, cntr,
                 sum2, sq2, sum3, sq3, t1, t2p, t3p):
    ph = pl.program_id(0)
    i = pl.program_id(1)
    col = pl.ds(i * _RB, _RB)

    @pl.when(jnp.logical_and(ph == 0, i == 0))
    def _zero():
        for r in (gram1, cntr, sum2, sq2, sum3, sq3):
            r[...] = jnp.zeros_like(r)

    @pl.when(ph == 0)
    def _phase_a():
        acc_g = jnp.zeros((_C + 1, _C + 1), jnp.float32)
        for p in range(_NPTS):
            pp = poly_ref[p]
            mb = mask_ref[p].astype(jnp.bfloat16)
            a = jnp.concatenate([pp * mb, mb], axis=0)
            b = jnp.concatenate([pp, mb], axis=0)
            acc_g = acc_g + jax.lax.dot_general(
                a, b, (((1,), (1,)), ((), ())),
                preferred_element_type=jnp.float32)
        gram1[...] += acc_g

    @pl.when(ph == 1)
    def _phase_b():
        @pl.when(i == 0)
        def _():
            g10 = gram1[...]
            cnt = jnp.maximum(g10[_C, _C], 1.0)
            cntr[...] = jnp.full((1, 1), cnt, jnp.float32)
            w1 = w1_ref[...]
            w1f = w1.astype(jnp.float32)
            a2 = jnp.dot(w1, g10[:_C, :_C].astype(jnp.bfloat16),
                         preferred_element_type=jnp.float32)
            mean = jnp.sum(w1f * g10[_C:_C + 1, :_C], axis=1,
                           keepdims=True) / cnt
            q = jnp.sum(a2 * w1f, axis=1, keepdims=True)
            var = q / cnt - mean * mean
            inv = jax.lax.rsqrt(var + _EPS) * g1_ref[...]
            t1[...] = be1_ref[...] - mean * inv
            w1s_scr[...] = (w1f * inv).astype(jnp.bfloat16)

        w1s = w1s_scr[...]
        sh = t1[...]
        w2a = w2a_ref[...]
        pooled = None
        cacc = jnp.zeros((1, _RB), jnp.float32)
        acc_s = jnp.zeros((_H, _RB), jnp.float32)
        acc_q = jnp.zeros((_H, _RB), jnp.float32)
        for p in range(_NPTS):
            m = mask_ref[p]
            x = jnp.dot(w1s, poly_ref[p], preferred_element_type=jnp.float32)
            y = jnp.maximum(x + sh, 0.0) * m
            x2a = jnp.dot(w2a, y.astype(jnp.bfloat16),
                          preferred_element_type=jnp.float32)
            x2_scr[p, :, col] = x2a.astype(jnp.bfloat16)
            x2am = x2a * m
            acc_s = acc_s + x2am
            acc_q = acc_q + x2am * x2am
            pooled = y if p == 0 else jnp.maximum(pooled, y)
            cacc = cacc + m
        pb = jnp.dot(w2b_ref[...], pooled.astype(jnp.bfloat16),
                     preferred_element_type=jnp.float32)
        pb_scr[:, col] = pb.astype(jnp.bfloat16)
        adj_s = acc_s + pb * cacc
        adj_q = acc_q + (2.0 * acc_s + pb * cacc) * pb
        sum2[...] += jnp.sum(adj_s, axis=1, keepdims=True)
        sq2[...] += jnp.sum(adj_q, axis=1, keepdims=True)

    @pl.when(ph == 2)
    def _phase_c():
        @pl.when(i == 0)
        def _():
            cnt = cntr[0, 0]
            mean = sum2[...] / cnt
            var = sq2[...] / cnt - mean * mean
            inv = jax.lax.rsqrt(var + _EPS) * g2_ref[...]
            t2p[...] = (be2_ref[...] - mean * inv) / inv
            w3s_scr[...] = (w3_ref[...].astype(jnp.float32)
                            * jnp.swapaxes(inv, 0, 1)).astype(jnp.bfloat16)

        t2eff = pb_scr[:, col].astype(jnp.float32) + t2p[...]
        w3s = w3s_scr[...]
        acc_s = jnp.zeros((_H, _RB), jnp.float32)
        acc_q = jnp.zeros((_H, _RB), jnp.float32)
        for p in range(_NPTS):
            m = mask_ref[p]
            z = jnp.maximum(
                x2_scr[p, :, col].astype(jnp.float32) + t2eff, 0.0) * m
            x3 = jnp.dot(w3s, z.astype(jnp.bfloat16),
                         preferred_element_type=jnp.float32)
            x2_scr[p, :, col] = x3.astype(jnp.bfloat16)
            x3m = x3 * m
            acc_s = acc_s + x3m
            acc_q = acc_q + x3m * x3m
        sum3[...] += jnp.sum(acc_s, axis=1, keepdims=True)
        sq3[...] += jnp.sum(acc_q, axis=1, keepdims=True)

    @pl.when(ph == 3)
    def _phase_d():
        @pl.when(i == 0)
        def _():
            cnt = cntr[0, 0]
            mean = sum3[...] / cnt
            var = sq3[...] / cnt - mean * mean
            inv = jax.lax.rsqrt(var + _EPS) * g3_ref[...]
            t3p[...] = (be3_ref[...] - mean * inv) / inv
            w4s_scr[...] = (w4_ref[...].astype(jnp.float32)
                            * jnp.swapaxes(inv, 0, 1)).astype(jnp.bfloat16)

        sh = t3p[...]
        w4s = w4s_scr[...]
        buf = None
        v = None
        for p in range(_NPTS):
            m = mask_ref[p]
            h3 = jnp.maximum(
                x2_scr[p, :, col].astype(jnp.float32) + sh, 0.0) * m
            buf = h3 if p == 0 else jnp.maximum(buf, h3)
            v = m if p == 0 else jnp.maximum(v, m)
        o = jnp.maximum(
            jnp.dot(w4s, buf.astype(jnp.bfloat16),
                    preferred_element_type=jnp.float32) + b4_ref[...], 0.0)
        o = jnp.dot(w5_ref[...], o.astype(jnp.bfloat16),
                    preferred_element_type=jnp.float32) + b5_ref[...]
        out_ref[...] = jnp.swapaxes(o * v, 0, 1)


def kernel(polylines, polylines_mask, W1, g1, b1, W2, g2, b2, W3, g3, b3,
           W4, bo4, W5, bo5):
    B, NP, NPTS, C = polylines.shape
    BNP = B * NP

    poly = polylines.reshape(BNP, NPTS, C).transpose(1, 2, 0).astype(
        jnp.bfloat16)
    maskf = polylines_mask.reshape(BNP, NPTS).T[:, None, :].astype(jnp.float32)
    w1t = W1.T.astype(jnp.bfloat16)
    w2at = W2[:_H].T.astype(jnp.bfloat16)
    w2bt = W2[_H:].T.astype(jnp.bfloat16)
    w3t = W3.T.astype(jnp.bfloat16)
    w4t = W4.T.astype(jnp.bfloat16)
    w5t = W5.T.astype(jnp.bfloat16)

    full = lambda shape: pl.BlockSpec(shape, lambda p, i: tuple(
        0 for _ in shape))
    vec = lambda: pl.BlockSpec((_H, 1), lambda p, i: (0, 0))

    out = pl.pallas_call(
        _mega_kernel,
        grid=(4, _NBLK),
        in_specs=[
            pl.BlockSpec((NPTS, C, _RB),
                         lambda p, i: (0, 0, jax.lax.select(p < 2, i, 0))),
            pl.BlockSpec((NPTS, 1, _RB), lambda p, i: (0, 0, i)),
            full((_H, C)), full((_H, _H)), full((_H, _H)), full((_H, _H)),
            full((_H, _H)), vec(), full((_OUT, _H)),
            pl.BlockSpec((_OUT, 1), lambda p, i: (0, 0)),
            vec(), vec(), vec(), vec(), vec(), vec(),
        ],
        out_specs=pl.BlockSpec(
            (_RB, _OUT), lambda p, i: (jax.lax.select(p == 3, i, 0), 0)),
        out_shape=jax.ShapeDtypeStruct((BNP, _OUT), jnp.float32),
        scratch_shapes=[
            pltpu.VMEM((_NPTS, _H, BNP), jnp.bfloat16),
            pltpu.VMEM((_H, BNP), jnp.bfloat16),
            pltpu.VMEM((_H, _C), jnp.bfloat16),
            pltpu.VMEM((_H, _H), jnp.bfloat16),
            pltpu.VMEM((_H, _H), jnp.bfloat16),
            pltpu.VMEM((_C + 1, _C + 1), jnp.float32),
            pltpu.VMEM((1, 1), jnp.float32),
            pltpu.VMEM((_H, 1), jnp.float32),
            pltpu.VMEM((_H, 1), jnp.float32),
            pltpu.VMEM((_H, 1), jnp.float32),
            pltpu.VMEM((_H, 1), jnp.float32),
            pltpu.VMEM((_H, 1), jnp.float32),
            pltpu.VMEM((_H, 1), jnp.float32),
            pltpu.VMEM((_H, 1), jnp.float32),
        ],
        compiler_params=pltpu.CompilerParams(
            dimension_semantics=("arbitrary", "arbitrary")),
    )(poly, maskf, w1t, w2at, w2bt, w3t, w4t,
      bo4[:, None].astype(jnp.float32), w5t, bo5[:, None].astype(jnp.float32),
      g1[:, None], b1[:, None], g2[:, None], b2[:, None], g3[:, None],
      b3[:, None])
    return out.reshape(B, NP, _OUT)


# speedup vs baseline: 1.0219x; 1.0219x over previous
import jax
import jax.numpy as jnp
from jax.experimental import pallas as pl
from jax.experimental.pallas import tpu as pltpu

_EPS = 1e-5
_NPTS = 20
_C = 9
_H = 64
_OUT = 128
_RB = 2048
_NBLK = 16384 // _RB


def _mega_kernel(poly_ref, mask_ref, w1_ref, w2a_ref, w2b_ref,
                 w3_ref, w4_ref, b4_ref, w5_ref, b5_ref, g1_ref, be1_ref,
                 g2_ref, be2_ref, g3_ref, be3_ref, out_ref,
                 x2_scr, pb_scr, w1s_scr, w3s_scr, w4s_scr, gram1, cntr,
                 sum2, sq2, sum3, sq3, t1, t2p, t3p):
    ph = pl.program_id(0)
    i = pl.program_id(1)
    col = pl.ds(i * _RB, _RB)

    @pl.when(jnp.logical_and(ph == 0, i == 0))
    def _zero():
        for r in (gram1, cntr, sum2, sq2, sum3, sq3):
            r[...] = jnp.zeros_like(r)

    @pl.when(ph == 0)
    def _phase_a():
        acc_g = jnp.zeros((_C + 1, _C + 1), jnp.float32)
        for p in range(_NPTS):
            pp = poly_ref[p]
            mb = mask_ref[p].astype(jnp.bfloat16)
            a = jnp.concatenate([pp * mb, mb], axis=0)
            b = jnp.concatenate([pp, mb], axis=0)
            acc_g = acc_g + jax.lax.dot_general(
                a, b, (((1,), (1,)), ((), ())),
                preferred_element_type=jnp.float32)
        gram1[...] += acc_g

    @pl.when(ph == 1)
    def _phase_b():
        @pl.when(i == 0)
        def _():
            g10 = gram1[...]
            cnt = jnp.maximum(g10[_C, _C], 1.0)
            cntr[...] = jnp.full((1, 1), cnt, jnp.float32)
            w1 = w1_ref[...]
            w1f = w1.astype(jnp.float32)
            a2 = jnp.dot(w1, g10[:_C, :_C].astype(jnp.bfloat16),
                         preferred_element_type=jnp.float32)
            mean = jnp.sum(w1f * g10[_C:_C + 1, :_C], axis=1,
                           keepdims=True) / cnt
            q = jnp.sum(a2 * w1f, axis=1, keepdims=True)
            var = q / cnt - mean * mean
            inv = jax.lax.rsqrt(var + _EPS) * g1_ref[...]
            t1[...] = be1_ref[...] - mean * inv
            w1s_scr[...] = (w1f * inv).astype(jnp.bfloat16)

        w1s = w1s_scr[...]
        sh = t1[...]
        w2a = w2a_ref[...]
        pooled = None
        cacc = jnp.zeros((1, _RB), jnp.float32)
        acc_s = jnp.zeros((_H, _RB), jnp.float32)
        gq = jnp.zeros((_H, _H), jnp.float32)
        for p in range(_NPTS):
            m = mask_ref[p]
            x = jnp.dot(w1s, poly_ref[p], preferred_element_type=jnp.float32)
            y = jnp.maximum(x + sh, 0.0) * m
            x2a = jnp.dot(w2a, y.astype(jnp.bfloat16),
                          preferred_element_type=jnp.float32)
            x2_scr[p, :, col] = x2a.astype(jnp.bfloat16)
            x2am = x2a * m
            acc_s = acc_s + x2am
            xmb = x2am.astype(jnp.bfloat16)
            gq = gq + jax.lax.dot_general(
                xmb, xmb, (((1,), (1,)), ((), ())),
                preferred_element_type=jnp.float32)
            pooled = y if p == 0 else jnp.maximum(pooled, y)
            cacc = cacc + m
        pb = jnp.dot(w2b_ref[...], pooled.astype(jnp.bfloat16),
                     preferred_element_type=jnp.float32)
        pb_scr[:, col] = pb.astype(jnp.bfloat16)
        adj_s = acc_s + pb * cacc
        adj_q = (2.0 * acc_s + pb * cacc) * pb
        eye = (jax.lax.broadcasted_iota(jnp.int32, (_H, _H), 0)
               == jax.lax.broadcasted_iota(jnp.int32, (_H, _H), 1)
               ).astype(jnp.float32)
        sum2[...] += jnp.sum(adj_s, axis=1, keepdims=True)
        sq2[...] += (jnp.sum(gq * eye, axis=1, keepdims=True)
                     + jnp.sum(adj_q, axis=1, keepdims=True))

    @pl.when(ph == 2)
    def _phase_c():
        @pl.when(i == 0)
        def _():
            cnt = cntr[0, 0]
            mean = sum2[...] / cnt
            var = sq2[...] / cnt - mean * mean
            inv = jax.lax.rsqrt(var + _EPS) * g2_ref[...]
            t2p[...] = (be2_ref[...] - mean * inv) / inv
            w3s_scr[...] = (w3_ref[...].astype(jnp.float32)
                            * jnp.swapaxes(inv, 0, 1)).astype(jnp.bfloat16)

        t2eff = pb_scr[:, col].astype(jnp.float32) + t2p[...]
        w3s = w3s_scr[...]
        acc_s = jnp.zeros((_H, _RB), jnp.float32)
        gq = jnp.zeros((_H, _H), jnp.float32)
        for p in range(_NPTS):
            m = mask_ref[p]
            z = jnp.maximum(
                x2_scr[p, :, col].astype(jnp.float32) + t2eff, 0.0) * m
            x3 = jnp.dot(w3s, z.astype(jnp.bfloat16),
                         preferred_element_type=jnp.float32)
            x2_scr[p, :, col] = x3.astype(jnp.bfloat16)
            x3m = x3 * m
            acc_s = acc_s + x3m
            xmb = x3m.astype(jnp.bfloat16)
            gq = gq + jax.lax.dot_general(
                xmb, xmb, (((1,), (1,)), ((), ())),
                preferred_element_type=jnp.float32)
        eye = (jax.lax.broadcasted_iota(jnp.int32, (_H, _H), 0)
               == jax.lax.broadcasted_iota(jnp.int32, (_H, _H), 1)
               ).astype(jnp.float32)
        sum3[...] += jnp.sum(acc_s, axis=1, keepdims=True)
        sq3[...] += jnp.sum(gq * eye, axis=1, keepdims=True)

    @pl.when(ph == 3)
    def _phase_d():
        @pl.when(i == 0)
        def _():
            cnt = cntr[0, 0]
            mean = sum3[...] / cnt
            var = sq3[...] / cnt - mean * mean
            inv = jax.lax.rsqrt(var + _EPS) * g3_ref[...]
            t3p[...] = (be3_ref[...] - mean * inv) / inv
            w4s_scr[...] = (w4_ref[...].astype(jnp.float32)
                            * jnp.swapaxes(inv, 0, 1)).astype(jnp.bfloat16)

        sh = t3p[...]
        w4s = w4s_scr[...]
        buf = None
        v = None
        for p in range(_NPTS):
            m = mask_ref[p]
            h3 = jnp.maximum(
                x2_scr[p, :, col].astype(jnp.float32) + sh, 0.0) * m
            buf = h3 if p == 0 else jnp.maximum(buf, h3)
            v = m if p == 0 else jnp.maximum(v, m)
        o = jnp.maximum(
            jnp.dot(w4s, buf.astype(jnp.bfloat16),
                    preferred_element_type=jnp.float32) + b4_ref[...], 0.0)
        o = jnp.dot(w5_ref[...], o.astype(jnp.bfloat16),
                    preferred_element_type=jnp.float32) + b5_ref[...]
        out_ref[...] = jnp.swapaxes(o * v, 0, 1)


def kernel(polylines, polylines_mask, W1, g1, b1, W2, g2, b2, W3, g3, b3,
           W4, bo4, W5, bo5):
    B, NP, NPTS, C = polylines.shape
    BNP = B * NP

    poly = polylines.reshape(BNP, NPTS, C).transpose(1, 2, 0).astype(
        jnp.bfloat16)
    maskf = polylines_mask.reshape(BNP, NPTS).T[:, None, :].astype(jnp.float32)
    w1t = W1.T.astype(jnp.bfloat16)
    w2at = W2[:_H].T.astype(jnp.bfloat16)
    w2bt = W2[_H:].T.astype(jnp.bfloat16)
    w3t = W3.T.astype(jnp.bfloat16)
    w4t = W4.T.astype(jnp.bfloat16)
    w5t = W5.T.astype(jnp.bfloat16)

    full = lambda shape: pl.BlockSpec(shape, lambda p, i: tuple(
        0 for _ in shape))
    vec = lambda: pl.BlockSpec((_H, 1), lambda p, i: (0, 0))

    out = pl.pallas_call(
        _mega_kernel,
        grid=(4, _NBLK),
        in_specs=[
            pl.BlockSpec((NPTS, C, _RB),
                         lambda p, i: (0, 0, jax.lax.select(p < 2, i, 0))),
            pl.BlockSpec((NPTS, 1, _RB), lambda p, i: (0, 0, i)),
            full((_H, C)), full((_H, _H)), full((_H, _H)), full((_H, _H)),
            full((_H, _H)), vec(), full((_OUT, _H)),
            pl.BlockSpec((_OUT, 1), lambda p, i: (0, 0)),
            vec(), vec(), vec(), vec(), vec(), vec(),
        ],
        out_specs=pl.BlockSpec(
            (_RB, _OUT), lambda p, i: (jax.lax.select(p == 3, i, 0), 0)),
        out_shape=jax.ShapeDtypeStruct((BNP, _OUT), jnp.float32),
        scratch_shapes=[
            pltpu.VMEM((_NPTS, _H, BNP), jnp.bfloat16),
            pltpu.VMEM((_H, BNP), jnp.bfloat16),
            pltpu.VMEM((_H, _C), jnp.bfloat16),
            pltpu.VMEM((_H, _H), jnp.bfloat16),
            pltpu.VMEM((_H, _H), jnp.bfloat16),
            pltpu.VMEM((_C + 1, _C + 1), jnp.float32),
            pltpu.VMEM((1, 1), jnp.float32),
            pltpu.VMEM((_H, 1), jnp.float32),
            pltpu.VMEM((_H, 1), jnp.float32),
            pltpu.VMEM((_H, 1), jnp.float32),
            pltpu.VMEM((_H, 1), jnp.float32),
            pltpu.VMEM((_H, 1), jnp.float32),
            pltpu.VMEM((_H, 1), jnp.float32),
            pltpu.VMEM((_H, 1), jnp.float32),
        ],
        compiler_params=pltpu.CompilerParams(
            dimension_semantics=("arbitrary", "arbitrary")),
    )(poly, maskf, w1t, w2at, w2bt, w3t, w4t,
      bo4[:, None].astype(jnp.float32), w5t, bo5[:, None].astype(jnp.float32),
      g1[:, None], b1[:, None], g2[:, None], b2[:, None], g3[:, None],
      b3[:, None])
    return out.reshape(B, NP, _OUT)
